# baseline (speedup 1.0000x reference)
import jax
import jax.numpy as jnp
from jax import lax
from jax.experimental import pallas as pl
from jax.experimental.pallas import tpu as pltpu
from jax.experimental.pallas import tpu_sc as plsc

B = 16384
D = 64
L = 16
NC = 2
NS = 16
NW = NC * NS
B_PER_W = B // NW
CHUNK = 128
NCHUNK = B_PER_W // CHUNK


def _body(users, items, user_table, item_table, out,
          idx_u, idx_i, rows_u, rows_i, out_v, sem_u, sem_i):
  wid = lax.axis_index("s") * NC + lax.axis_index("c")
  base = wid * B_PER_W

  for c in range(NCHUNK):
    pltpu.sync_copy(users.at[pl.ds(base + c * CHUNK, CHUNK)], idx_u.at[c])
    pltpu.sync_copy(items.at[pl.ds(base + c * CHUNK, CHUNK)], idx_i.at[c])

  copies = []
  for c in range(NCHUNK):
    sl = pl.ds(c * CHUNK, CHUNK)
    copies.append(
        (pltpu.async_copy(user_table.at[idx_u.at[c]], rows_u.at[sl], sem_u),
         pltpu.async_copy(item_table.at[idx_i.at[c]], rows_i.at[sl], sem_i)))

  riota = lax.iota(jnp.int32, L)

  for c in range(NCHUNK):
    cp_u, cp_i = copies[c]
    cp_u.wait()
    cp_i.wait()

    def block(j, carry, c=c):
      ro = c * CHUNK + j * L
      row_ids = riota + ro
      acc = jnp.zeros((L,), jnp.float32)
      for k in range(D):
        col = jnp.full((L,), k, jnp.int32)
        uv = plsc.load_gather(rows_u, [row_ids, col])
        iv = plsc.load_gather(rows_i, [row_ids, col])
        acc = acc + uv * iv
      out_v[pl.ds(ro, L)] = acc
      return carry

    lax.fori_loop(0, CHUNK // L, block, 0)

  pltpu.sync_copy(out_v, out.at[pl.ds(base, B_PER_W)])


@jax.jit
def kernel(users, items, user_table, item_table):
  mesh = plsc.VectorSubcoreMesh(core_axis_name="c", subcore_axis_name="s")
  k = pl.kernel(
      _body,
      out_type=jax.ShapeDtypeStruct((B,), jnp.float32),
      mesh=mesh,
      scratch_types=[
          pltpu.VMEM((NCHUNK, CHUNK), jnp.int32),
          pltpu.VMEM((NCHUNK, CHUNK), jnp.int32),
          pltpu.VMEM((B_PER_W, D), jnp.float32),
          pltpu.VMEM((B_PER_W, D), jnp.float32),
          pltpu.VMEM((B_PER_W,), jnp.float32),
          pltpu.SemaphoreType.DMA,
          pltpu.SemaphoreType.DMA,
      ],
      compiler_params=pltpu.CompilerParams(
          needs_layout_passes=False, use_tc_tiling_on_sc=False),
  )
  return k(users, items, user_table, item_table)


# speedup vs baseline: 1.5545x; 1.5545x over previous
import jax
import jax.numpy as jnp
from jax import lax
from jax.experimental import pallas as pl
from jax.experimental.pallas import tpu as pltpu
from jax.experimental.pallas import tpu_sc as plsc

B = 16384
D = 64
L = 16
NC = 2
NS = 16
NW = NC * NS
B_PER_W = B // NW
PASS_ROWS = B_PER_W // 2


def _body(users, items, user_table, item_table, out,
          idx_u_v, idx_i_v, rows_u, rows_i, out_v, sem_g):
  wid = lax.axis_index("s") * NC + lax.axis_index("c")
  base = wid * B_PER_W

  pltpu.sync_copy(users.at[pl.ds(base, B_PER_W)], idx_u_v)
  pltpu.sync_copy(items.at[pl.ds(base, B_PER_W)], idx_i_v)

  riota = lax.iota(jnp.int32, L)

  for p in range(2):
    poff = p * PASS_ROWS

    def issue(g, carry, poff=poff):
      uvec = idx_u_v[pl.ds(poff + g * L, L)]
      ivec = idx_i_v[pl.ds(poff + g * L, L)]
      for l in range(L):
        pltpu.async_copy(user_table.at[uvec[l]], rows_u.at[g * L + l], sem_g)
        pltpu.async_copy(item_table.at[ivec[l]], rows_i.at[g * L + l], sem_g)
      return carry

    lax.fori_loop(0, PASS_ROWS // L, issue, 0)

    def drain(g, carry, poff=poff):
      uvec = idx_u_v[pl.ds(poff + g * L, L)]
      ivec = idx_i_v[pl.ds(poff + g * L, L)]
      for l in range(L):
        pltpu.make_async_copy(
            user_table.at[uvec[l]], rows_u.at[g * L + l], sem_g).wait()
        pltpu.make_async_copy(
            item_table.at[ivec[l]], rows_i.at[g * L + l], sem_g).wait()
      return carry

    lax.fori_loop(0, PASS_ROWS // L, drain, 0)

    def block(j, carry):
      ro = j * L
      row_ids = riota + ro
      acc = jnp.zeros((L,), jnp.float32)
      for k in range(D):
        col = jnp.full((L,), k, jnp.int32)
        uv = plsc.load_gather(rows_u, [row_ids, col])
        iv = plsc.load_gather(rows_i, [row_ids, col])
        acc = acc + uv * iv
      out_v[pl.ds(ro, L)] = acc
      return carry

    lax.fori_loop(0, PASS_ROWS // L, block, 0)

    pltpu.sync_copy(out_v, out.at[pl.ds(base + poff, PASS_ROWS)])


@jax.jit
def kernel(users, items, user_table, item_table):
  mesh = plsc.VectorSubcoreMesh(core_axis_name="c", subcore_axis_name="s")
  k = pl.kernel(
      _body,
      out_type=jax.ShapeDtypeStruct((B,), jnp.float32),
      mesh=mesh,
      scratch_types=[
          pltpu.VMEM((B_PER_W,), jnp.int32),
          pltpu.VMEM((B_PER_W,), jnp.int32),
          pltpu.VMEM((PASS_ROWS, D), jnp.float32),
          pltpu.VMEM((PASS_ROWS, D), jnp.float32),
          pltpu.VMEM((PASS_ROWS,), jnp.float32),
          pltpu.SemaphoreType.DMA,
      ],
      compiler_params=pltpu.CompilerParams(needs_layout_passes=False),
  )
  return k(users, items, user_table, item_table)


# speedup vs baseline: 1.5601x; 1.0036x over previous
import jax
import jax.numpy as jnp
from jax import lax
from jax.experimental import pallas as pl
from jax.experimental.pallas import tpu as pltpu
from jax.experimental.pallas import tpu_sc as plsc

B = 16384
D = 64
L = 16
NC = 2
NS = 16
NW = NC * NS
B_PER_W = B // NW
PASS_ROWS = B_PER_W // 2


def _body(users, items, user_table, item_table, out,
          idx_u_v, idx_i_v, rows_u, rows_i, out_v, sem_g):
  wid = lax.axis_index("s") * NC + lax.axis_index("c")
  base = wid * B_PER_W

  with jax.named_scope("idx_stage"):
    pltpu.sync_copy(users.at[pl.ds(base, B_PER_W)], idx_u_v)
    pltpu.sync_copy(items.at[pl.ds(base, B_PER_W)], idx_i_v)

  riota = lax.iota(jnp.int32, L)

  for p in range(2):
    poff = p * PASS_ROWS

    with jax.named_scope("issue"):
      def issue(g, carry, poff=poff):
        uvec = idx_u_v[pl.ds(poff + g * L, L)]
        ivec = idx_i_v[pl.ds(poff + g * L, L)]
        for l in range(L):
          pltpu.async_copy(user_table.at[uvec[l]], rows_u.at[g * L + l],
                           sem_g)
          pltpu.async_copy(item_table.at[ivec[l]], rows_i.at[g * L + l],
                           sem_g)
        return carry

      lax.fori_loop(0, PASS_ROWS // L, issue, 0)

    with jax.named_scope("drain"):
      def drain(r, carry):
        pltpu.make_async_copy(user_table.at[0], rows_u.at[0], sem_g).wait()
        pltpu.make_async_copy(item_table.at[0], rows_i.at[0], sem_g).wait()
        return carry

      lax.fori_loop(0, PASS_ROWS, drain, 0)

    with jax.named_scope("compute"):
      def block(j, carry):
        ro = j * L
        row_ids = riota + ro
        acc = jnp.zeros((L,), jnp.float32)
        for k in range(D):
          col = jnp.full((L,), k, jnp.int32)
          uv = plsc.load_gather(rows_u, [row_ids, col])
          iv = plsc.load_gather(rows_i, [row_ids, col])
          acc = acc + uv * iv
        out_v[pl.ds(ro, L)] = acc
        return carry

      lax.fori_loop(0, PASS_ROWS // L, block, 0)

    pltpu.sync_copy(out_v, out.at[pl.ds(base + poff, PASS_ROWS)])


@jax.jit
def kernel(users, items, user_table, item_table):
  mesh = plsc.VectorSubcoreMesh(core_axis_name="c", subcore_axis_name="s")
  k = pl.kernel(
      _body,
      out_type=jax.ShapeDtypeStruct((B,), jnp.float32),
      mesh=mesh,
      scratch_types=[
          pltpu.VMEM((B_PER_W,), jnp.int32),
          pltpu.VMEM((B_PER_W,), jnp.int32),
          pltpu.VMEM((PASS_ROWS, D), jnp.float32),
          pltpu.VMEM((PASS_ROWS, D), jnp.float32),
          pltpu.VMEM((PASS_ROWS,), jnp.float32),
          pltpu.SemaphoreType.DMA,
      ],
      compiler_params=pltpu.CompilerParams(needs_layout_passes=False),
  )
  return k(users, items, user_table, item_table)
